# baseline (speedup 1.0000x reference)
import jax
import jax.numpy as jnp
from jax.experimental import pallas as pl
from jax.experimental.pallas import tpu as pltpu

_S = 4096
_D = 4096
_E = 64
_CAP = 64
_BS = 512
_GRID = _S // _BS


def _top1_kernel(x_ref, wgt_ref, laux_ref, combine_ref, loc_ref,
                 counts_ref, me_ref, ce_ref):
    i = pl.program_id(0)

    @pl.when(i == 0)
    def _init():
        counts_ref[...] = jnp.zeros_like(counts_ref)
        me_ref[...] = jnp.zeros_like(me_ref)
        ce_ref[...] = jnp.zeros_like(ce_ref)

    logits = jax.lax.dot_general(
        x_ref[...], wgt_ref[...], (((1,), (0,)), ((), ())),
        preferred_element_type=jnp.float32)
    gates = jax.nn.softmax(logits, axis=1)
    idx = jnp.argmax(gates, axis=1, keepdims=True)
    e_iota = jax.lax.broadcasted_iota(jnp.int32, (_BS, _E), 1)
    mask1 = (e_iota == idx).astype(jnp.float32)

    r_iota = jax.lax.broadcasted_iota(jnp.int32, (_BS, _BS), 0)
    c_iota = jax.lax.broadcasted_iota(jnp.int32, (_BS, _BS), 1)
    tri = (c_iota <= r_iota).astype(jnp.float32)
    csum = jax.lax.dot_general(
        tri, mask1, (((1,), (0,)), ((), ())),
        preferred_element_type=jnp.float32)

    counts = counts_ref[...]
    locations1 = csum - 1.0 + counts
    counts_ref[...] = counts + jnp.sum(mask1, axis=0, keepdims=True)

    me_ref[...] += jnp.sum(gates, axis=0, keepdims=True)
    ce_ref[...] += jnp.sum(mask1, axis=0, keepdims=True)

    loc = jnp.sum(locations1 * mask1, axis=1, keepdims=True)
    gate_s = jnp.sum(gates * mask1, axis=1, keepdims=True)
    val = jnp.where(loc < _CAP, gate_s, 0.0)

    idx_t = jnp.transpose(idx)
    loc_t = jnp.transpose(loc.astype(jnp.int32))
    val_t = jnp.transpose(val)
    e_iota_t = jax.lax.broadcasted_iota(jnp.int32, (_E, _BS), 0)
    one_e = (e_iota_t == idx_t)
    one_c = (e_iota_t == loc_t)
    a_t = one_e.astype(jnp.float32) * val_t
    b_t = one_c.astype(jnp.float32)

    combine_ref[...] = a_t[:, None, :] * b_t[None, :, :]
    loc_ref[...] = jnp.where(loc_t < _CAP, idx_t * _CAP + loc_t, -1)

    @pl.when(i == _GRID - 1)
    def _fin():
        me = me_ref[...] * (1.0 / _S)
        ce = ce_ref[...] * (1.0 / _S)
        laux_ref[0, 0] = jnp.mean(me * ce) * (_E * _E)


def kernel(input_tensor, wg):
    laux, combine_t, loc_v = pl.pallas_call(
        _top1_kernel,
        grid=(_GRID,),
        in_specs=[
            pl.BlockSpec((_BS, _D), lambda i: (i, 0)),
            pl.BlockSpec((_D, _E), lambda i: (0, 0)),
        ],
        out_specs=[
            pl.BlockSpec((1, 1), lambda i: (0, 0), memory_space=pltpu.SMEM),
            pl.BlockSpec((_E, _CAP, _BS), lambda i: (0, 0, i)),
            pl.BlockSpec((1, _BS), lambda i: (0, i)),
        ],
        out_shape=[
            jax.ShapeDtypeStruct((1, 1), jnp.float32),
            jax.ShapeDtypeStruct((_E, _CAP, _S), jnp.float32),
            jax.ShapeDtypeStruct((1, _S), jnp.int32),
        ],
        compiler_params=pltpu.CompilerParams(
            vmem_limit_bytes=100 * 1024 * 1024),
        scratch_shapes=[
            pltpu.VMEM((1, _E), jnp.float32),
            pltpu.VMEM((1, _E), jnp.float32),
            pltpu.VMEM((1, _E), jnp.float32),
        ],
    )(input_tensor, wg.T)
    combine = jnp.transpose(combine_t, (2, 0, 1))
    k_io = jax.lax.broadcasted_iota(jnp.int32, (_E * _CAP, _S), 0)
    dispatch_t = (k_io == loc_v).reshape(_E, _CAP, _S)
    dispatch = jnp.transpose(dispatch_t, (2, 0, 1))
    return laux[0, 0], combine, dispatch


# speedup vs baseline: 1.0296x; 1.0296x over previous
import jax
import jax.numpy as jnp
from jax.experimental import pallas as pl
from jax.experimental.pallas import tpu as pltpu

_S = 4096
_D = 4096
_E = 64
_CAP = 64
_BS = 512
_GRID = _S // _BS


def _top1_kernel(x_ref, wgt_ref, laux_ref, combine_ref, idx_ref, loc_ref,
                 counts_ref, me_ref):
    i = pl.program_id(0)

    @pl.when(i == 0)
    def _init():
        counts_ref[...] = jnp.zeros_like(counts_ref)
        me_ref[...] = jnp.zeros_like(me_ref)

    logits = jax.lax.dot_general(
        x_ref[...], wgt_ref[...], (((1,), (0,)), ((), ())),
        preferred_element_type=jnp.float32)
    gates = jax.nn.softmax(logits, axis=1)
    idx = jnp.argmax(gates, axis=1, keepdims=True)
    e_iota = jax.lax.broadcasted_iota(jnp.int32, (_BS, _E), 1)
    mask1 = (e_iota == idx).astype(jnp.float32)

    r_iota = jax.lax.broadcasted_iota(jnp.int32, (_BS, _BS), 0)
    c_iota = jax.lax.broadcasted_iota(jnp.int32, (_BS, _BS), 1)
    tri = (c_iota <= r_iota).astype(jnp.float32)
    csum = jax.lax.dot_general(
        tri, mask1, (((1,), (0,)), ((), ())),
        preferred_element_type=jnp.float32)

    counts = counts_ref[...]
    locations1 = csum - 1.0 + counts
    counts_ref[...] = counts + jnp.sum(mask1, axis=0, keepdims=True)

    me_ref[...] += jnp.sum(gates, axis=0, keepdims=True)

    loc = jnp.sum(locations1 * mask1, axis=1, keepdims=True)
    gate_s = jnp.sum(gates * mask1, axis=1, keepdims=True)
    val = jnp.where(loc < _CAP, gate_s, 0.0)

    idx_t = jnp.transpose(idx)
    loc_t = jnp.transpose(loc.astype(jnp.int32))
    val_t = jnp.transpose(val)
    e_iota_t = jax.lax.broadcasted_iota(jnp.int32, (_E, _BS), 0)
    one_e = (e_iota_t == idx_t)
    one_c = (e_iota_t == loc_t)
    a_t = one_e.astype(jnp.float32) * val_t
    b_t = one_c.astype(jnp.float32)

    combine_ref[...] = a_t[:, None, :] * b_t[None, :, :]
    idx_ref[...] = idx_t
    loc_ref[...] = loc_t

    @pl.when(i == _GRID - 1)
    def _fin():
        me = me_ref[...] * (1.0 / _S)
        ce = counts_ref[...] * (1.0 / _S)
        laux_ref[0, 0] = jnp.mean(me * ce) * (_E * _E)


def kernel(input_tensor, wg):
    laux, combine_t, idx_v, loc_v = pl.pallas_call(
        _top1_kernel,
        grid=(_GRID,),
        in_specs=[
            pl.BlockSpec((_BS, _D), lambda i: (i, 0)),
            pl.BlockSpec((_D, _E), lambda i: (0, 0)),
        ],
        out_specs=[
            pl.BlockSpec((1, 1), lambda i: (0, 0), memory_space=pltpu.SMEM),
            pl.BlockSpec((_E, _CAP, _BS), lambda i: (0, 0, i)),
            pl.BlockSpec((1, _BS), lambda i: (0, i)),
            pl.BlockSpec((1, _BS), lambda i: (0, i)),
        ],
        out_shape=[
            jax.ShapeDtypeStruct((1, 1), jnp.float32),
            jax.ShapeDtypeStruct((_E, _CAP, _S), jnp.float32),
            jax.ShapeDtypeStruct((1, _S), jnp.int32),
            jax.ShapeDtypeStruct((1, _S), jnp.int32),
        ],
        scratch_shapes=[
            pltpu.VMEM((1, _E), jnp.float32),
            pltpu.VMEM((1, _E), jnp.float32),
        ],
    )(input_tensor, wg.T)
    combine = jnp.transpose(combine_t, (2, 0, 1))
    e_io = jax.lax.broadcasted_iota(jnp.int32, (_E, _CAP, _S), 0)
    c_io = jax.lax.broadcasted_iota(jnp.int32, (_E, _CAP, _S), 1)
    dispatch_t = (e_io == idx_v[None, :, :]) & (c_io == loc_v[None, :, :])
    dispatch = jnp.transpose(dispatch_t, (2, 0, 1))
    return laux[0, 0], combine, dispatch
